# baseline (speedup 1.0000x reference)
import functools

import jax
import jax.numpy as jnp
from jax import lax
from jax.experimental import pallas as pl
from jax.experimental.pallas import tpu as pltpu
from jax.experimental.pallas import tpu_sc as plsc

N_NODES = 10000
N_EDGES = 320000
D_FEAT = 128
DH = D_FEAT // 2

NC = 2
NS = 16
CHUNK = 128
NBUF = 4
EDGES_PER_TILE = N_EDGES // NS
NCHUNK = 160
EPT_PAD = NCHUNK * CHUNK
N_PAD = 10240
ROWS_PER_SUB = N_PAD // NS
STAGE_ROWS = N_NODES // NS
NGRP = NCHUNK // NBUF


@functools.partial(
    pl.kernel,
    out_type=jax.ShapeDtypeStruct((NC, N_PAD, DH), jnp.float32),
    mesh=plsc.VectorSubcoreMesh(core_axis_name="c", subcore_axis_name="s"),
    compiler_params=pltpu.CompilerParams(use_tc_tiling_on_sc=False),
    scratch_types=[
        pltpu.VMEM((2, NBUF, CHUNK), jnp.int32),
        pltpu.VMEM((2, NBUF, CHUNK), jnp.int32),
        pltpu.VMEM((2, NBUF, CHUNK), jnp.float32),
        pltpu.VMEM((NBUF, CHUNK, DH), jnp.float32),
        pltpu.VMEM_SHARED((N_NODES, DH), jnp.float32),
        pltpu.VMEM_SHARED((N_PAD, DH), jnp.float32),
        [pltpu.SemaphoreType.DMA] * NBUF,
        pltpu.SemaphoreType.DMA,
        pltpu.SemaphoreType.DMA,
    ],
)
def _sc_spmm(xt2_hbm, col_hbm, row_hbm, val_hbm, part_hbm,
             col_q, row_q, val_q, rows_v, xsh, acc, gsem, isem, tsem):
    cid = lax.axis_index("c")
    sid = lax.axis_index("s")

    pltpu.async_copy(xt2_hbm.at[cid].at[pl.ds(sid * STAGE_ROWS, STAGE_ROWS)],
                     xsh.at[pl.ds(sid * STAGE_ROWS, STAGE_ROWS)], tsem)

    def zbody(r, carry):
        for j in range(DH // 16):
            rows_v[0, r, pl.ds(j * 16, 16)] = jnp.zeros((16,), jnp.float32)
        return carry

    lax.fori_loop(0, CHUNK, zbody, 0)
    for t in range(ROWS_PER_SUB // CHUNK):
        pltpu.sync_copy(rows_v.at[0],
                        acc.at[pl.ds(sid * ROWS_PER_SUB + t * CHUNK, CHUNK)])

    pltpu.make_async_copy(
        xt2_hbm.at[cid].at[pl.ds(0, STAGE_ROWS)],
        xsh.at[pl.ds(sid * STAGE_ROWS, STAGE_ROWS)], tsem).wait()
    plsc.subcore_barrier()

    def fetch_idx(q, qb):
        sl = pl.ds(q * NBUF, NBUF)
        pltpu.async_copy(col_hbm.at[sid, sl], col_q.at[qb], isem)
        pltpu.async_copy(row_hbm.at[sid, sl], row_q.at[qb], isem)
        pltpu.async_copy(val_hbm.at[sid, sl], val_q.at[qb], isem)

    def wait_idx(qb):
        pltpu.make_async_copy(col_hbm.at[sid, pl.ds(0, NBUF)],
                              col_q.at[qb], isem).wait()
        pltpu.make_async_copy(row_hbm.at[sid, pl.ds(0, NBUF)],
                              row_q.at[qb], isem).wait()
        pltpu.make_async_copy(val_hbm.at[sid, pl.ds(0, NBUF)],
                              val_q.at[qb], isem).wait()

    def scale(qb, b):
        def grp(g, c2):
            vv = val_q[qb, b, pl.ds(g * 16, 16)]
            base = g * 16
            for i in range(16):
                v = vv[i]
                for j in range(DH // 16):
                    sl = pl.ds(j * 16, 16)
                    rows_v[b, base + i, sl] = rows_v[b, base + i, sl] * v
            return c2

        lax.fori_loop(0, CHUNK // 16, grp, 0)

    fetch_idx(0, 0)
    wait_idx(0)
    for b in range(NBUF):
        pltpu.async_copy(xsh.at[col_q.at[0, b]], rows_v.at[b], gsem[b])
    fetch_idx(1, 1)

    def group(p, carry):
        qb = lax.rem(p, 2)
        qn = 1 - qb

        @pl.when(p + 1 < NGRP)
        def _():
            wait_idx(qn)

        for b in range(NBUF):
            pltpu.make_async_copy(xsh.at[col_q.at[qb, b]], rows_v.at[b],
                                  gsem[b]).wait()
            scale(qb, b)
            pltpu.sync_copy(rows_v.at[b], acc.at[row_q.at[qb, b]], add=True)

            @pl.when(p + 1 < NGRP)
            def _():
                pltpu.async_copy(xsh.at[col_q.at[qn, b]], rows_v.at[b],
                                 gsem[b])

        @pl.when(p + 2 < NGRP)
        def _():
            fetch_idx(p + 2, qb)

        return carry

    lax.fori_loop(0, NGRP, group, 0)
    plsc.subcore_barrier()

    base = sid * ROWS_PER_SUB
    pltpu.sync_copy(acc.at[pl.ds(base, ROWS_PER_SUB)],
                    part_hbm.at[cid].at[pl.ds(base, ROWS_PER_SUB)])


_BN = 1024


def _merge_body(p_ref, o_ref):
    o_ref[...] = jnp.concatenate([p_ref[0].T, p_ref[1].T], axis=0)


_merge = pl.pallas_call(
    _merge_body,
    grid=(N_PAD // _BN,),
    in_specs=[pl.BlockSpec((NC, _BN, DH), lambda i: (0, i, 0))],
    out_specs=pl.BlockSpec((D_FEAT, _BN), lambda i: (0, i)),
    out_shape=jax.ShapeDtypeStruct((D_FEAT, N_NODES), jnp.float32),
)


def kernel(x, synset_indices, synset_values):
    xt = x.T
    xt2 = jnp.stack([xt[:, :DH], xt[:, DH:]])
    pad = EPT_PAD - EDGES_PER_TILE
    row = synset_indices[0].reshape(NS, EDGES_PER_TILE)
    col = synset_indices[1].reshape(NS, EDGES_PER_TILE)
    val = synset_values.reshape(NS, EDGES_PER_TILE)
    row = jnp.pad(row, ((0, 0), (0, pad))).reshape(NS, NCHUNK, CHUNK)
    col = jnp.pad(col, ((0, 0), (0, pad))).reshape(NS, NCHUNK, CHUNK)
    val = jnp.pad(val, ((0, 0), (0, pad))).reshape(NS, NCHUNK, CHUNK)
    part = _sc_spmm(xt2, col, row, val)
    return _merge(part)


# speedup vs baseline: 1.1202x; 1.1202x over previous
import functools

import jax
import jax.numpy as jnp
from jax import lax
from jax.experimental import pallas as pl
from jax.experimental.pallas import tpu as pltpu
from jax.experimental.pallas import tpu_sc as plsc

N_NODES = 10000
N_EDGES = 320000
D_FEAT = 128
DH = D_FEAT // 2

NC = 2
NS = 16
CHUNK = 128
NBUF = 5
EDGES_PER_TILE = N_EDGES // NS
NCHUNK = 160
EPT_PAD = NCHUNK * CHUNK
N_PAD = 10240
ROWS_PER_SUB = N_PAD // NS
STAGE_ROWS = N_NODES // NS
NGRP = NCHUNK // NBUF


@functools.partial(
    pl.kernel,
    out_type=jax.ShapeDtypeStruct((NC, N_PAD, DH), jnp.float32),
    mesh=plsc.VectorSubcoreMesh(core_axis_name="c", subcore_axis_name="s"),
    compiler_params=pltpu.CompilerParams(use_tc_tiling_on_sc=False),
    scratch_types=[
        pltpu.VMEM((2, NBUF, CHUNK), jnp.int32),
        pltpu.VMEM((2, NBUF, CHUNK), jnp.int32),
        pltpu.VMEM((2, NBUF, CHUNK), jnp.float32),
        pltpu.VMEM((NBUF, CHUNK, DH), jnp.float32),
        pltpu.VMEM_SHARED((N_NODES, DH), jnp.float32),
        pltpu.VMEM_SHARED((N_PAD, DH), jnp.float32),
        [pltpu.SemaphoreType.DMA] * NBUF,
        [pltpu.SemaphoreType.DMA] * NBUF,
        pltpu.SemaphoreType.DMA,
        pltpu.SemaphoreType.DMA,
    ],
)
def _sc_spmm(xt2_hbm, col_hbm, row_hbm, val_hbm, part_hbm,
             col_q, row_q, val_q, rows_v, xsh, acc, gsem, ssem, isem, tsem):
    cid = lax.axis_index("c")
    sid = lax.axis_index("s")

    pltpu.async_copy(xt2_hbm.at[cid].at[pl.ds(sid * STAGE_ROWS, STAGE_ROWS)],
                     xsh.at[pl.ds(sid * STAGE_ROWS, STAGE_ROWS)], tsem)

    def zbody(r, carry):
        for j in range(DH // 16):
            rows_v[0, r, pl.ds(j * 16, 16)] = jnp.zeros((16,), jnp.float32)
        return carry

    lax.fori_loop(0, CHUNK, zbody, 0)
    for t in range(ROWS_PER_SUB // CHUNK):
        pltpu.sync_copy(rows_v.at[0],
                        acc.at[pl.ds(sid * ROWS_PER_SUB + t * CHUNK, CHUNK)])

    pltpu.make_async_copy(
        xt2_hbm.at[cid].at[pl.ds(0, STAGE_ROWS)],
        xsh.at[pl.ds(sid * STAGE_ROWS, STAGE_ROWS)], tsem).wait()
    plsc.subcore_barrier()

    def fetch_idx(q, qb):
        sl = pl.ds(q * NBUF, NBUF)
        pltpu.async_copy(col_hbm.at[sid, sl], col_q.at[qb], isem)
        pltpu.async_copy(row_hbm.at[sid, sl], row_q.at[qb], isem)
        pltpu.async_copy(val_hbm.at[sid, sl], val_q.at[qb], isem)

    def wait_idx(qb):
        pltpu.make_async_copy(col_hbm.at[sid, pl.ds(0, NBUF)],
                              col_q.at[qb], isem).wait()
        pltpu.make_async_copy(row_hbm.at[sid, pl.ds(0, NBUF)],
                              row_q.at[qb], isem).wait()
        pltpu.make_async_copy(val_hbm.at[sid, pl.ds(0, NBUF)],
                              val_q.at[qb], isem).wait()

    def scale(qb, b):
        def grp(g, c2):
            vv = val_q[qb, b, pl.ds(g * 16, 16)]
            base = g * 16
            for i in range(16):
                v = vv[i]
                for j in range(DH // 16):
                    sl = pl.ds(j * 16, 16)
                    rows_v[b, base + i, sl] = rows_v[b, base + i, sl] * v
            return c2

        lax.fori_loop(0, CHUNK // 16, grp, 0)

    fetch_idx(0, 0)
    wait_idx(0)
    for b in range(NBUF):
        pltpu.async_copy(xsh.at[col_q.at[0, b]], rows_v.at[b], gsem[b])
    fetch_idx(1, 1)

    def group(p, carry):
        qb = lax.rem(p, 2)
        qn = 1 - qb

        @pl.when(p + 1 < NGRP)
        def _():
            wait_idx(qn)

        for b in range(NBUF):
            pltpu.make_async_copy(xsh.at[col_q.at[qb, b]], rows_v.at[b],
                                  gsem[b]).wait()
            scale(qb, b)
            pltpu.async_copy(rows_v.at[b], acc.at[row_q.at[qb, b]], ssem[b],
                             add=True)

        for b in range(NBUF):
            pltpu.make_async_copy(rows_v.at[b], acc.at[row_q.at[qb, b]],
                                  ssem[b]).wait()

            @pl.when(p + 1 < NGRP)
            def _():
                pltpu.async_copy(xsh.at[col_q.at[qn, b]], rows_v.at[b],
                                 gsem[b])

        @pl.when(p + 2 < NGRP)
        def _():
            fetch_idx(p + 2, qb)

        return carry

    lax.fori_loop(0, NGRP, group, 0)
    plsc.subcore_barrier()

    base = sid * ROWS_PER_SUB
    pltpu.sync_copy(acc.at[pl.ds(base, ROWS_PER_SUB)],
                    part_hbm.at[cid].at[pl.ds(base, ROWS_PER_SUB)])


_BN = 1024


def _merge_body(p_ref, o_ref):
    o_ref[...] = jnp.concatenate([p_ref[0].T, p_ref[1].T], axis=0)


_merge = pl.pallas_call(
    _merge_body,
    grid=(N_PAD // _BN,),
    in_specs=[pl.BlockSpec((NC, _BN, DH), lambda i: (0, i, 0))],
    out_specs=pl.BlockSpec((D_FEAT, _BN), lambda i: (0, i)),
    out_shape=jax.ShapeDtypeStruct((D_FEAT, N_NODES), jnp.float32),
)


def kernel(x, synset_indices, synset_values):
    xt = x.T
    xt2 = jnp.stack([xt[:, :DH], xt[:, DH:]])
    pad = EPT_PAD - EDGES_PER_TILE
    row = synset_indices[0].reshape(NS, EDGES_PER_TILE)
    col = synset_indices[1].reshape(NS, EDGES_PER_TILE)
    val = synset_values.reshape(NS, EDGES_PER_TILE)
    row = jnp.pad(row, ((0, 0), (0, pad))).reshape(NS, NCHUNK, CHUNK)
    col = jnp.pad(col, ((0, 0), (0, pad))).reshape(NS, NCHUNK, CHUNK)
    val = jnp.pad(val, ((0, 0), (0, pad))).reshape(NS, NCHUNK, CHUNK)
    part = _sc_spmm(xt2, col, row, val)
    return _merge(part)
